# baseline (speedup 1.0000x reference)
import functools
import math

import jax
import jax.numpy as jnp
from jax import lax
from jax.experimental import pallas as pl
from jax.experimental.pallas import tpu as pltpu
from jax.experimental.pallas import tpu_sc as plsc

_SCALE_POSE = 10.0
_SCALE_RADAR_V = 5.0
_L_MIN = 2 * math.log(0.03 / _SCALE_POSE + 1e-09)
_L_MAX = 2 * math.log(0.5 / _SCALE_POSE + 1e-09)
_R_MIN = 2 * math.log(0.1 / _SCALE_RADAR_V + 1e-09)
_R_MAX = 2 * math.log(5.0 / _SCALE_RADAR_V + 1e-09)
_GHOST = (0.6 / _SCALE_POSE) ** 2

_NL = 100000
_EL = 1600000
_NR = 20000
_NRP = 20480
_ETP = 128000
_ECP = 256000
_NGT = 256
_D = 8
_CH = 2000

_mesh = plsc.VectorSubcoreMesh(core_axis_name="c", subcore_axis_name="s")
_sc_params = pltpu.CompilerParams(use_tc_tiling_on_sc=False)
_sc_params_nl = pltpu.CompilerParams(
    use_tc_tiling_on_sc=False, needs_layout_passes=False)


def _pipelined_pass(n, base0, ei0, ei1, table, sink, src_v, dst_v, rows_v,
                    sem_i, sem_g, sem_s, write_linear=False, gout=None):
    def idx_copies(k):
        b = k % 2
        ops = [(ei0.at[0, pl.ds(base0 + k * _CH, _CH)] if ei0.ndim == 2
                else ei0.at[pl.ds(base0 + k * _CH, _CH)], src_v[b], sem_i[b])]
        if not write_linear:
            ops.append((ei1.at[1, pl.ds(base0 + k * _CH, _CH)] if ei1.ndim == 2
                        else ei1.at[pl.ds(base0 + k * _CH, _CH)],
                        dst_v[b], sem_i[b]))
        return ops

    def start_idx(k):
        for s_, d_, m_ in idx_copies(k):
            pltpu.async_copy(s_, d_, m_)

    def wait_idx(k):
        for s_, d_, m_ in idx_copies(k):
            pltpu.make_async_copy(s_, d_, m_).wait()

    def gather_args(k):
        b = k % 2
        return table.at[src_v[b]], rows_v[b], sem_g[b]

    def sink_args(k):
        b = k % 2
        if write_linear:
            return rows_v[b], gout.at[pl.ds(base0 + k * _CH, _CH)], sem_s[b]
        return rows_v[b], sink.at[dst_v[b]], sem_s[b]

    start_idx(0)
    for k in range(n):
        wait_idx(k)
        pltpu.async_copy(*gather_args(k))
        if k + 1 < n:
            if k >= 1:
                pltpu.make_async_copy(*sink_args(k - 1)).wait()
            start_idx(k + 1)
        elif k >= 1:
            pltpu.make_async_copy(*sink_args(k - 1)).wait()
        pltpu.make_async_copy(*gather_args(k)).wait()
        if write_linear:
            pltpu.async_copy(*sink_args(k))
        else:
            s_, d_, m_ = sink_args(k)
            pltpu.async_copy(s_, d_, m_, add=True)
    pltpu.make_async_copy(*sink_args(n - 1)).wait()


@functools.partial(
    pl.kernel,
    out_type=(
        jax.ShapeDtypeStruct((2, _NL, _D), jnp.float32),
        jax.ShapeDtypeStruct((2, _NRP, _D), jnp.float32),
        jax.ShapeDtypeStruct((2, _NRP, _D), jnp.float32),
        jax.ShapeDtypeStruct((_ETP, _D), jnp.float32),
        jax.ShapeDtypeStruct((_ETP, _D), jnp.float32),
        jax.ShapeDtypeStruct((_ETP, _D), jnp.float32),
        jax.ShapeDtypeStruct((_ETP, _D), jnp.float32),
    ),
    scratch_types=[
        pltpu.VMEM((_CH,), jnp.int32),
        pltpu.VMEM((_CH,), jnp.int32),
        pltpu.VMEM((_CH,), jnp.int32),
        pltpu.VMEM((_CH,), jnp.int32),
        pltpu.VMEM((_CH, _D), jnp.float32),
        pltpu.VMEM((_CH, _D), jnp.float32),
        pltpu.VMEM_SHARED((_NL, _D), jnp.float32),
        pltpu.VMEM_SHARED((_NRP, _D), jnp.float32),
        pltpu.VMEM_SHARED((_NRP, _D), jnp.float32),
        pltpu.SemaphoreType.DMA,
        pltpu.SemaphoreType.DMA,
        pltpu.SemaphoreType.DMA,
        pltpu.SemaphoreType.DMA,
        pltpu.SemaphoreType.DMA,
        pltpu.SemaphoreType.DMA,
    ],
    mesh=_mesh,
    compiler_params=_sc_params,
)
def _sc_edge_pass(t_lid, t_r1, t_r2, lid_ei, r1cs, r1cd, r2cs, r2cd, tei1,
                  tei2, zeros, lid_acc, r1_acc, r2_acc, g1s, g1d, g2s, g2d,
                  src_v0, src_v1, dst_v0, dst_v1, rows_v0, rows_v1,
                  accl, acc1, acc2, semi0, semi1, semg0, semg1, sems0, sems1):
    c = lax.axis_index("c")
    s = lax.axis_index("s")
    wid = c * 16 + s
    src_v = (src_v0, src_v1)
    dst_v = (dst_v0, dst_v1)
    rows_v = (rows_v0, rows_v1)
    sem_i = (semi0, semi1)
    sem_g = (semg0, semg1)
    sem_s = (sems0, sems1)

    nl16 = _NL // 16
    nr16 = _NRP // 16
    pltpu.sync_copy(zeros.at[pl.ds(s * nl16, nl16)], accl.at[pl.ds(s * nl16, nl16)])
    pltpu.sync_copy(zeros.at[pl.ds(s * nr16, nr16)], acc1.at[pl.ds(s * nr16, nr16)])
    pltpu.sync_copy(zeros.at[pl.ds(s * nr16, nr16)], acc2.at[pl.ds(s * nr16, nr16)])
    plsc.subcore_barrier()

    common = dict(src_v=src_v, dst_v=dst_v, rows_v=rows_v,
                  sem_i=sem_i, sem_g=sem_g, sem_s=sem_s)
    _pipelined_pass(_EL // 32 // _CH, wid * (_EL // 32), lid_ei, lid_ei,
                    t_lid, accl, **common)
    for cs_ref, cd_ref, acc in ((r1cs, r1cd, acc1), (r2cs, r2cd, acc2)):
        _pipelined_pass(_ECP // 32 // _CH, wid * (_ECP // 32), cd_ref, cs_ref,
                        t_lid, acc, **common)
    for tei, t_r, gs, gd in ((tei1, t_r1, g1s, g1d), (tei2, t_r2, g2s, g2d)):
        _pipelined_pass(_ETP // 32 // _CH, wid * (_ETP // 32), tei, None,
                        t_r, None, write_linear=True, gout=gs, **common)
        tei_dst = tei.at[1]
        _pipelined_pass(_ETP // 32 // _CH, wid * (_ETP // 32), tei_dst, None,
                        t_r, None, write_linear=True, gout=gd, **common)

    plsc.subcore_barrier()
    pltpu.sync_copy(accl.at[pl.ds(s * nl16, nl16)], lid_acc.at[c, pl.ds(s * nl16, nl16)])
    pltpu.sync_copy(acc1.at[pl.ds(s * nr16, nr16)], r1_acc.at[c, pl.ds(s * nr16, nr16)])
    pltpu.sync_copy(acc2.at[pl.ds(s * nr16, nr16)], r2_acc.at[c, pl.ds(s * nr16, nr16)])


@functools.partial(
    pl.kernel,
    out_type=jax.ShapeDtypeStruct((1, 16), jnp.float32),
    scratch_types=[
        pltpu.VMEM((_CH,), jnp.int32),
        pltpu.VMEM((_CH,), jnp.float32),
        pltpu.VMEM((_NR,), jnp.int32),
        pltpu.VMEM((_NR,), jnp.float32),
        pltpu.VMEM((16,), jnp.float32),
        pltpu.VMEM((2, 16), jnp.float32),
        pltpu.VMEM((1, 16), jnp.float32),
        pltpu.VMEM((1, 16), jnp.float32),
        pltpu.VMEM((1, 16), jnp.float32),
        pltpu.VMEM_SHARED((2, 16), jnp.float32),
    ],
    mesh=_mesh,
    compiler_params=_sc_params_nl,
)
def _sc_phys_pass(tei1, md1, den1, tei2, md2, den2, sl, sr1, sr2, out,
                  src_v, md_v, eid_v, den_v, ovec, ph_v, sl_v, s1_v, s2_v,
                  phys_sp):
    c = lax.axis_index("c")
    s = lax.axis_index("s")
    ne = 100000

    def run_branch(tei, md, den, row):
        zi = jnp.zeros((16,), jnp.int32)
        def init_body(i, _):
            eid_v[pl.ds(i * 16, 16)] = zi
            return 0
        lax.fori_loop(0, _NR // 16, init_body, 0)
        pltpu.sync_copy(den, den_v)
        lanes = lax.iota(jnp.int32, 16)

        def p1_chunk(k, _):
            pltpu.sync_copy(tei.at[0, pl.ds(k * _CH, _CH)], src_v)
            def p1_vreg(j, _):
                idx = src_v[pl.ds(j * 16, 16)]
                my = (k * _CH + j * 16 + 1) + lanes
                plsc.store_scatter(eid_v, [idx], my)
                def rnd(r, _):
                    g = plsc.load_gather(eid_v, [idx])
                    m = my > g
                    @pl.when(jnp.any(m))
                    def _():
                        plsc.store_scatter(eid_v, [idx], my, mask=m)
                    return 0
                lax.fori_loop(0, 3, rnd, 0)
                return 0
            lax.fori_loop(0, _CH // 16, p1_vreg, 0)
            return 0
        lax.fori_loop(0, ne // _CH, p1_chunk, 0)

        def p2_chunk(k, acc):
            pltpu.sync_copy(tei.at[0, pl.ds(k * _CH, _CH)], src_v)
            pltpu.sync_copy(md.at[pl.ds(k * _CH, _CH)], md_v)
            def p2_vreg(j, acc):
                idx = src_v[pl.ds(j * 16, 16)]
                my = (k * _CH + j * 16 + 1) + lanes
                g = plsc.load_gather(eid_v, [idx])
                w = g == my
                dg = plsc.load_gather(den_v, [idx])
                mdv = md_v[pl.ds(j * 16, 16)]
                return acc + jnp.where(w, mdv / dg, 0.0)
            return lax.fori_loop(0, _CH // 16, p2_vreg, acc)
        acc = lax.fori_loop(0, ne // _CH, p2_chunk, jnp.zeros((16,), jnp.float32))
        ovec[...] = acc
        pltpu.sync_copy(ovec, phys_sp.at[row])

    @pl.when(jnp.logical_and(c == 0, s == 0))
    def _():
        run_branch(tei1, md1, den1, 0)

    @pl.when(jnp.logical_and(c == 0, s == 1))
    def _():
        run_branch(tei2, md2, den2, 1)

    plsc.subcore_barrier()

    @pl.when(jnp.logical_and(c == 0, s == 0))
    def _():
        pltpu.sync_copy(phys_sp, ph_v)
        pltpu.sync_copy(sl, sl_v)
        pltpu.sync_copy(sr1, s1_v)
        pltpu.sync_copy(sr2, s2_v)
        def bsum(v):
            return jnp.broadcast_to(jnp.sum(v), (16,))
        nl = jnp.full((16,), float(_NL), jnp.float32)
        nr = jnp.full((16,), float(_NR), jnp.float32)
        tv = (bsum(sl_v[0]) / nl
              + (bsum(s1_v[0]) + bsum(ph_v[0])) / nr
              + (bsum(s2_v[0]) + bsum(ph_v[1])) / nr)
        lanes = lax.iota(jnp.int32, 16)
        ovec[...] = jnp.where(lanes == 0, tv, 0.0)
        pltpu.sync_copy(ovec, out.at[0])


def _k0_body(lpos_ref, lx_ref, rpos_ref, rx_ref, rb_ref, dt_ref,
             tlid_ref, trad_ref):
    pos = lpos_ref[...]
    x2 = lx_ref[:, 2:3]
    sq = jnp.sum(pos * pos, axis=1, keepdims=True)
    ones = jnp.ones_like(x2)
    z = jnp.zeros_like(pos[:, 0:2])
    tlid_ref[...] = jnp.concatenate([pos, x2, sq, ones, z], axis=1)

    rpos = rpos_ref[...]
    sp = jnp.abs(rx_ref[:, 2:3])
    b = rb_ref[...]
    nd = jnp.zeros_like(sp)
    for bb in range(8):
        nd = jnp.where(b == bb, dt_ref[0:1, bb:bb + 1], nd)
    nd = jnp.maximum(nd, 0.01)
    rz = jnp.zeros_like(rpos)
    trad_ref[...] = jnp.concatenate([rpos, sp, nd, rz], axis=1)


def _b_radar(a0, a1, p, ro, ndt):
    acc = a0 + a1
    s2 = acc[:, 0:3]
    s1 = acc[:, 4:5]
    cnt = acc[:, 5:6]
    rsq = jnp.sum(p * p, axis=1, keepdims=True)
    sum_d = cnt * rsq - 2.0 * jnp.sum(p * s2, axis=1, keepdims=True) + s1
    val = sum_d / jnp.maximum(cnt, 1.0) ** 2
    spat = jnp.where(cnt > 0, val, _GHOST)
    lv = jnp.clip(ro, _R_MIN, _R_MAX)
    den = 2.0 * jnp.exp(lv) * ndt * ndt + 1e-9
    return den, jnp.sum(spat / den + 0.5 * lv)


def _b_mind2(gs, gd, g, b2):
    ps = gs[:, 0:3]
    sp = gs[:, 3:4]
    nd = gs[:, 4:5]
    pd = gd[:, 0:3]
    mv = pd - ps
    nrm = jnp.sqrt(jnp.sum(mv * mv, axis=1, keepdims=True))
    unit = mv / (nrm + 1e-9)
    pred = ps + sp * unit * nd
    a2 = jnp.sum(pred * pred, axis=1, keepdims=True)
    d2 = jnp.maximum(
        a2 + b2 - 2.0 * jnp.dot(pred, g.T, preferred_element_type=jnp.float32),
        0.0)
    return jnp.min(d2, axis=1, keepdims=True)


def _b_body(la0_ref, la1_ref, lpos_ref, lx_ref, lo_ref,
            r1a0_ref, r1a1_ref, r1pos_ref, r1o_ref, tr1_ref,
            r2a0_ref, r2a1_ref, r2pos_ref, r2o_ref, tr2_ref,
            g1s_ref, g1d_ref, g2s_ref, g2d_ref, gt_ref,
            md1_ref, md2_ref, den1_ref, den2_ref,
            sl_ref, s1_ref, s2_ref):
    i = pl.program_id(0)

    acc = la0_ref[...] + la1_ref[...]
    cnt = acc[:, 5:6]
    dc = jnp.maximum(cnt, 1.0)
    mp = acc[:, 0:3] / dc
    mi = acc[:, 3:4] / dc
    p = lpos_ref[...]
    res_pos = jnp.sum((p - mp) ** 2, axis=1, keepdims=True)
    res_int = (lx_ref[:, 2:3] - mi) ** 2
    lv = jnp.clip(lo_ref[...], _L_MIN, _L_MAX)
    prec = jnp.exp(-lv)
    tot_l = jnp.sum(0.5 * prec * res_pos + 0.5 * prec * res_int + 0.5 * lv)

    den1, tot_1 = _b_radar(r1a0_ref[...], r1a1_ref[...], r1pos_ref[...],
                           r1o_ref[...], tr1_ref[:, 4:5])
    den2, tot_2 = _b_radar(r2a0_ref[...], r2a1_ref[...], r2pos_ref[...],
                           r2o_ref[...], tr2_ref[:, 4:5])
    den1_ref[...] = den1
    den2_ref[...] = den2

    g = gt_ref[...]
    b2 = jnp.sum(g * g, axis=1)[None, :]
    md1_ref[...] = _b_mind2(g1s_ref[...], g1d_ref[...], g, b2)
    md2_ref[...] = _b_mind2(g2s_ref[...], g2d_ref[...], g, b2)

    iota16 = lax.broadcasted_iota(jnp.int32, (1, 16), 1)
    @pl.when(i == 0)
    def _():
        sl_ref[...] = jnp.zeros((1, 16), jnp.float32)
        s1_ref[...] = jnp.zeros((1, 16), jnp.float32)
        s2_ref[...] = jnp.zeros((1, 16), jnp.float32)
    sl_ref[...] += jnp.where(iota16 == 0, tot_l, 0.0)
    s1_ref[...] += jnp.where(iota16 == 0, tot_1, 0.0)
    s2_ref[...] += jnp.where(iota16 == 0, tot_2, 0.0)


def _blk(shape, imap):
    return pl.BlockSpec(shape, imap)


def kernel(lidar_out, lidar_pos, lidar_x, lidar_spatial_edge_index, radar1_out,
           radar1_pos, radar1_x, radar1_batch, radar1_temporal_edge_index,
           radar1_to_lidar_src, radar1_to_lidar_dst, radar2_out, radar2_pos,
           radar2_x, radar2_batch, radar2_temporal_edge_index,
           radar2_to_lidar_src, radar2_to_lidar_dst, dt_sec, gt_radar_pos):
    f32, i32 = jnp.float32, jnp.int32

    lid_ei = lidar_spatial_edge_index.astype(i32)
    pad_c = _ECP - 200000
    pad_t = _ETP - 100000
    cpad_s = _NR + (jnp.arange(pad_c, dtype=i32) % (_NRP - _NR))
    cpad_d = jnp.arange(pad_c, dtype=i32) % _NL
    r1cs = jnp.concatenate([radar1_to_lidar_src.astype(i32), cpad_s])
    r1cd = jnp.concatenate([radar1_to_lidar_dst.astype(i32), cpad_d])
    r2cs = jnp.concatenate([radar2_to_lidar_src.astype(i32), cpad_s])
    r2cd = jnp.concatenate([radar2_to_lidar_dst.astype(i32), cpad_d])
    tpad = jnp.broadcast_to(jnp.arange(pad_t, dtype=i32) % _NR, (2, pad_t))
    tei1 = jnp.concatenate([radar1_temporal_edge_index.astype(i32), tpad], axis=1)
    tei2 = jnp.concatenate([radar2_temporal_edge_index.astype(i32), tpad], axis=1)
    zeros = jnp.zeros((_NL, _D), f32)
    dt2 = dt_sec.reshape(1, 8).astype(f32)

    rpos = jnp.concatenate([radar1_pos, radar2_pos], axis=0)
    rx = jnp.concatenate([radar1_x, radar2_x], axis=0)
    rb = jnp.concatenate([radar1_batch.astype(i32),
                          radar2_batch.astype(i32)]).reshape(2 * _NR, 1)
    t_lid, t_rad = pl.pallas_call(
        _k0_body,
        grid=(100,),
        in_specs=[_blk((1000, 3), lambda i: (i, 0)),
                  _blk((1000, 3), lambda i: (i, 0)),
                  _blk((400, 3), lambda i: (i, 0)),
                  _blk((400, 3), lambda i: (i, 0)),
                  _blk((400, 1), lambda i: (i, 0)),
                  _blk((1, 8), lambda i: (0, 0))],
        out_specs=[_blk((1000, _D), lambda i: (i, 0)),
                   _blk((400, _D), lambda i: (i, 0))],
        out_shape=[jax.ShapeDtypeStruct((_NL, _D), f32),
                   jax.ShapeDtypeStruct((2 * _NR, _D), f32)],
    )(lidar_pos, lidar_x, rpos, rx, rb, dt2)
    t_r1 = t_rad[:_NR]
    t_r2 = t_rad[_NR:]

    lid_acc, r1_acc, r2_acc, g1s, g1d, g2s, g2d = _sc_edge_pass(
        t_lid, t_r1, t_r2, lid_ei, r1cs, r1cd, r2cs, r2cd, tei1, tei2, zeros)

    LB, RB, TB = 800, 160, 1024
    (md1, md2, den1, den2, s_lid, s_r1, s_r2) = pl.pallas_call(
        _b_body,
        grid=(125,),
        in_specs=[_blk((LB, _D), lambda i: (i, 0)),
                  _blk((LB, _D), lambda i: (i, 0)),
                  _blk((LB, 3), lambda i: (i, 0)),
                  _blk((LB, 3), lambda i: (i, 0)),
                  _blk((LB, 1), lambda i: (i, 0)),
                  _blk((RB, _D), lambda i: (i, 0)),
                  _blk((RB, _D), lambda i: (i, 0)),
                  _blk((RB, 3), lambda i: (i, 0)),
                  _blk((RB, 1), lambda i: (i, 0)),
                  _blk((RB, _D), lambda i: (i, 0)),
                  _blk((RB, _D), lambda i: (i, 0)),
                  _blk((RB, _D), lambda i: (i, 0)),
                  _blk((RB, 3), lambda i: (i, 0)),
                  _blk((RB, 1), lambda i: (i, 0)),
                  _blk((RB, _D), lambda i: (i, 0)),
                  _blk((TB, _D), lambda i: (i, 0)),
                  _blk((TB, _D), lambda i: (i, 0)),
                  _blk((TB, _D), lambda i: (i, 0)),
                  _blk((TB, _D), lambda i: (i, 0)),
                  _blk((_NGT, 3), lambda i: (0, 0))],
        out_specs=[_blk((TB, 1), lambda i: (i, 0)),
                   _blk((TB, 1), lambda i: (i, 0)),
                   _blk((RB, 1), lambda i: (i, 0)),
                   _blk((RB, 1), lambda i: (i, 0)),
                   _blk((1, 16), lambda i: (0, 0)),
                   _blk((1, 16), lambda i: (0, 0)),
                   _blk((1, 16), lambda i: (0, 0))],
        out_shape=[jax.ShapeDtypeStruct((_ETP, 1), f32),
                   jax.ShapeDtypeStruct((_ETP, 1), f32),
                   jax.ShapeDtypeStruct((_NR, 1), f32),
                   jax.ShapeDtypeStruct((_NR, 1), f32),
                   jax.ShapeDtypeStruct((1, 16), f32),
                   jax.ShapeDtypeStruct((1, 16), f32),
                   jax.ShapeDtypeStruct((1, 16), f32)],
    )(lid_acc[0], lid_acc[1], lidar_pos, lidar_x, lidar_out,
      r1_acc[0, :_NR], r1_acc[1, :_NR], radar1_pos, radar1_out, t_r1,
      r2_acc[0, :_NR], r2_acc[1, :_NR], radar2_pos, radar2_out, t_r2,
      g1s, g1d, g2s, g2d, gt_radar_pos)

    tot = _sc_phys_pass(tei1, md1.reshape(_ETP), den1.reshape(_NR),
                        tei2, md2.reshape(_ETP), den2.reshape(_NR),
                        s_lid, s_r1, s_r2)
    return tot[0, 0]


# speedup vs baseline: 1.4488x; 1.4488x over previous
import functools
import math

import jax
import jax.numpy as jnp
from jax import lax
from jax.experimental import pallas as pl
from jax.experimental.pallas import tpu as pltpu
from jax.experimental.pallas import tpu_sc as plsc

_SCALE_POSE = 10.0
_SCALE_RADAR_V = 5.0
_L_MIN = 2 * math.log(0.03 / _SCALE_POSE + 1e-09)
_L_MAX = 2 * math.log(0.5 / _SCALE_POSE + 1e-09)
_R_MIN = 2 * math.log(0.1 / _SCALE_RADAR_V + 1e-09)
_R_MAX = 2 * math.log(5.0 / _SCALE_RADAR_V + 1e-09)
_GHOST = (0.6 / _SCALE_POSE) ** 2

_NL = 100000
_EL = 1600000
_NR = 20000
_NRP = 20480
_ETP = 128000
_ECP = 256000
_NGT = 256
_D = 8
_CH = 2000

_mesh = plsc.VectorSubcoreMesh(core_axis_name="c", subcore_axis_name="s")
_sc_params = pltpu.CompilerParams(use_tc_tiling_on_sc=False)
_sc_params_nl = pltpu.CompilerParams(
    use_tc_tiling_on_sc=False, needs_layout_passes=False)


def _pipelined_pass(n, base0, ei0, ei1, table, sink, src_v, dst_v, rows_v,
                    sem_i, sem_g, sem_s, write_linear=False, gout=None):
    def idx_copies(k):
        b = k % 2
        ops = [(ei0.at[0, pl.ds(base0 + k * _CH, _CH)] if ei0.ndim == 2
                else ei0.at[pl.ds(base0 + k * _CH, _CH)], src_v[b], sem_i[b])]
        if not write_linear:
            ops.append((ei1.at[1, pl.ds(base0 + k * _CH, _CH)] if ei1.ndim == 2
                        else ei1.at[pl.ds(base0 + k * _CH, _CH)],
                        dst_v[b], sem_i[b]))
        return ops

    def start_idx(k):
        for s_, d_, m_ in idx_copies(k):
            pltpu.async_copy(s_, d_, m_)

    def wait_idx(k):
        for s_, d_, m_ in idx_copies(k):
            pltpu.make_async_copy(s_, d_, m_).wait()

    def gather_args(k):
        b = k % 2
        return table.at[src_v[b]], rows_v[b], sem_g[b]

    def sink_args(k):
        b = k % 2
        if write_linear:
            return rows_v[b], gout.at[pl.ds(base0 + k * _CH, _CH)], sem_s[b]
        return rows_v[b], sink.at[dst_v[b]], sem_s[b]

    start_idx(0)
    for k in range(n):
        wait_idx(k)
        pltpu.async_copy(*gather_args(k))
        if k + 1 < n:
            if k >= 1:
                pltpu.make_async_copy(*sink_args(k - 1)).wait()
            start_idx(k + 1)
        elif k >= 1:
            pltpu.make_async_copy(*sink_args(k - 1)).wait()
        pltpu.make_async_copy(*gather_args(k)).wait()
        if write_linear:
            pltpu.async_copy(*sink_args(k))
        else:
            s_, d_, m_ = sink_args(k)
            pltpu.async_copy(s_, d_, m_, add=True)
    pltpu.make_async_copy(*sink_args(n - 1)).wait()


@functools.partial(
    pl.kernel,
    out_type=(
        jax.ShapeDtypeStruct((2, _NL, _D), jnp.float32),
        jax.ShapeDtypeStruct((2, _NRP, _D), jnp.float32),
        jax.ShapeDtypeStruct((2, _NRP, _D), jnp.float32),
        jax.ShapeDtypeStruct((_ETP, _D), jnp.float32),
        jax.ShapeDtypeStruct((_ETP, _D), jnp.float32),
        jax.ShapeDtypeStruct((_ETP, _D), jnp.float32),
        jax.ShapeDtypeStruct((_ETP, _D), jnp.float32),
    ),
    scratch_types=[
        pltpu.VMEM((_CH,), jnp.int32),
        pltpu.VMEM((_CH,), jnp.int32),
        pltpu.VMEM((_CH,), jnp.int32),
        pltpu.VMEM((_CH,), jnp.int32),
        pltpu.VMEM((_CH, _D), jnp.float32),
        pltpu.VMEM((_CH, _D), jnp.float32),
        pltpu.VMEM_SHARED((_NL, _D), jnp.float32),
        pltpu.VMEM_SHARED((_NRP, _D), jnp.float32),
        pltpu.VMEM_SHARED((_NRP, _D), jnp.float32),
        pltpu.SemaphoreType.DMA,
        pltpu.SemaphoreType.DMA,
        pltpu.SemaphoreType.DMA,
        pltpu.SemaphoreType.DMA,
        pltpu.SemaphoreType.DMA,
        pltpu.SemaphoreType.DMA,
    ],
    mesh=_mesh,
    compiler_params=_sc_params,
)
def _sc_edge_pass(t_lid, t_r1, t_r2, lid_ei, r1cs, r1cd, r2cs, r2cd, tei1,
                  tei2, zeros, lid_acc, r1_acc, r2_acc, g1s, g1d, g2s, g2d,
                  src_v0, src_v1, dst_v0, dst_v1, rows_v0, rows_v1,
                  accl, acc1, acc2, semi0, semi1, semg0, semg1, sems0, sems1):
    c = lax.axis_index("c")
    s = lax.axis_index("s")
    wid = c * 16 + s
    src_v = (src_v0, src_v1)
    dst_v = (dst_v0, dst_v1)
    rows_v = (rows_v0, rows_v1)
    sem_i = (semi0, semi1)
    sem_g = (semg0, semg1)
    sem_s = (sems0, sems1)

    nl16 = _NL // 16
    nr16 = _NRP // 16
    pltpu.sync_copy(zeros, accl.at[pl.ds(s * nl16, nl16)])
    pltpu.sync_copy(zeros.at[pl.ds(0, nr16)], acc1.at[pl.ds(s * nr16, nr16)])
    pltpu.sync_copy(zeros.at[pl.ds(0, nr16)], acc2.at[pl.ds(s * nr16, nr16)])
    plsc.subcore_barrier()

    common = dict(src_v=src_v, dst_v=dst_v, rows_v=rows_v,
                  sem_i=sem_i, sem_g=sem_g, sem_s=sem_s)
    _pipelined_pass(_EL // 32 // _CH, wid * (_EL // 32), lid_ei, lid_ei,
                    t_lid, accl, **common)
    for cs_ref, cd_ref, acc in ((r1cs, r1cd, acc1), (r2cs, r2cd, acc2)):
        _pipelined_pass(_ECP // 32 // _CH, wid * (_ECP // 32), cd_ref, cs_ref,
                        t_lid, acc, **common)
    for tei, t_r, gs, gd in ((tei1, t_r1, g1s, g1d), (tei2, t_r2, g2s, g2d)):
        _pipelined_pass(_ETP // 32 // _CH, wid * (_ETP // 32), tei, None,
                        t_r, None, write_linear=True, gout=gs, **common)
        tei_dst = tei.at[1]
        _pipelined_pass(_ETP // 32 // _CH, wid * (_ETP // 32), tei_dst, None,
                        t_r, None, write_linear=True, gout=gd, **common)

    plsc.subcore_barrier()
    pltpu.sync_copy(accl.at[pl.ds(s * nl16, nl16)], lid_acc.at[c, pl.ds(s * nl16, nl16)])
    pltpu.sync_copy(acc1.at[pl.ds(s * nr16, nr16)], r1_acc.at[c, pl.ds(s * nr16, nr16)])
    pltpu.sync_copy(acc2.at[pl.ds(s * nr16, nr16)], r2_acc.at[c, pl.ds(s * nr16, nr16)])


_NEC = 50
_NRE = 20224
_MS = 1264

@functools.partial(
    pl.kernel,
    out_type=jax.ShapeDtypeStruct((2, 16), jnp.float32),
    scratch_types=[
        pltpu.VMEM((_CH,), jnp.int32),
        pltpu.VMEM((_CH,), jnp.float32),
        pltpu.VMEM((_NRE,), jnp.int32),
        pltpu.VMEM((_MS,), jnp.int32),
        pltpu.VMEM((_NR,), jnp.float32),
        pltpu.VMEM((16,), jnp.float32),
        pltpu.VMEM((16, 16), jnp.float32),
        pltpu.VMEM((1, 16), jnp.float32),
        pltpu.VMEM((1, 16), jnp.float32),
        pltpu.VMEM_SHARED((16, _NRE), jnp.int32),
        pltpu.VMEM_SHARED((_NRE,), jnp.int32),
        pltpu.VMEM_SHARED((16, 16), jnp.float32),
    ],
    mesh=_mesh,
    compiler_params=_sc_params_nl,
)
def _sc_phys_pass(tei_b, md_b, den_b, sl, sr1, sr2, out,
                  src_v, md_v, eid_v, tmp_v, den_v, ovec, ph_v, sa_v, sb_v,
                  eid_sp, merged_sp, phys_sp):
    c = lax.axis_index("c")
    s = lax.axis_index("s")
    lanes = lax.iota(jnp.int32, 16)

    zi = jnp.zeros((16,), jnp.int32)
    def init_body(i, _):
        eid_v[pl.ds(i * 16, 16)] = zi
        return 0
    lax.fori_loop(0, _NRE // 16, init_body, 0)

    def p1_chunk(k):
        pltpu.sync_copy(tei_b.at[c, 0, pl.ds(k * _CH, _CH)], src_v)
        def p1_vreg(j, _):
            idx = src_v[pl.ds(j * 16, 16)]
            my = (k * _CH + j * 16 + 1) + lanes
            plsc.store_scatter(eid_v, [idx], my)
            def rnd(r, _):
                g = plsc.load_gather(eid_v, [idx])
                m = my > g
                @pl.when(jnp.any(m))
                def _():
                    plsc.store_scatter(eid_v, [idx], my, mask=m)
                return 0
            lax.fori_loop(0, 3, rnd, 0)
            return 0
        lax.fori_loop(0, _CH // 16, p1_vreg, 0)

    for r in range((_NEC + 15) // 16):
        k = s + 16 * r
        @pl.when(k < _NEC)
        def _(k=k):
            p1_chunk(k)

    pltpu.sync_copy(eid_v, eid_sp.at[s])
    plsc.subcore_barrier()
    pltpu.sync_copy(eid_sp.at[0, pl.ds(s * _MS, _MS)], tmp_v)
    def cp_body(j, _):
        eid_v[pl.ds(j * 16, 16)] = tmp_v[pl.ds(j * 16, 16)]
        return 0
    lax.fori_loop(0, _MS // 16, cp_body, 0)
    for rr in range(1, 16):
        pltpu.sync_copy(eid_sp.at[rr, pl.ds(s * _MS, _MS)], tmp_v)
        def mx_body(j, _):
            eid_v[pl.ds(j * 16, 16)] = jnp.maximum(
                eid_v[pl.ds(j * 16, 16)], tmp_v[pl.ds(j * 16, 16)])
            return 0
        lax.fori_loop(0, _MS // 16, mx_body, 0)
    pltpu.sync_copy(eid_v.at[pl.ds(0, _MS)], merged_sp.at[pl.ds(s * _MS, _MS)])
    plsc.subcore_barrier()
    pltpu.sync_copy(merged_sp, eid_v)
    pltpu.sync_copy(den_b.at[c], den_v)

    ovec[...] = jnp.zeros((16,), jnp.float32)
    def p2_chunk(k):
        pltpu.sync_copy(tei_b.at[c, 0, pl.ds(k * _CH, _CH)], src_v)
        pltpu.sync_copy(md_b.at[c, pl.ds(k * _CH, _CH)], md_v)
        def p2_vreg(j, acc):
            idx = src_v[pl.ds(j * 16, 16)]
            my = (k * _CH + j * 16 + 1) + lanes
            g = plsc.load_gather(eid_v, [idx])
            w = g == my
            dg = plsc.load_gather(den_v, [idx])
            mdv = md_v[pl.ds(j * 16, 16)]
            return acc + jnp.where(w, mdv / dg, 0.0)
        acc = lax.fori_loop(0, _CH // 16, p2_vreg, jnp.zeros((16,), jnp.float32))
        ovec[...] += acc

    for r in range((_NEC + 15) // 16):
        k = s + 16 * r
        @pl.when(k < _NEC)
        def _(k=k):
            p2_chunk(k)

    pltpu.sync_copy(ovec, phys_sp.at[s])
    plsc.subcore_barrier()

    @pl.when(s == 0)
    def _():
        pltpu.sync_copy(phys_sp, ph_v)
        phs = jnp.zeros((16,), jnp.float32)
        for rr in range(16):
            phs = phs + ph_v[rr]
        def bsum(v):
            return jnp.broadcast_to(jnp.sum(v), (16,))
        nl = jnp.full((16,), float(_NL), jnp.float32)
        nr = jnp.full((16,), float(_NR), jnp.float32)

        @pl.when(c == 0)
        def _():
            pltpu.sync_copy(sl, sa_v)
            pltpu.sync_copy(sr1, sb_v)
            tv = bsum(sa_v[0]) / nl + (bsum(sb_v[0]) + bsum(phs)) / nr
            ovec[...] = jnp.where(lanes == 0, tv, 0.0)
            pltpu.sync_copy(ovec, out.at[0])

        @pl.when(c == 1)
        def _():
            pltpu.sync_copy(sr2, sb_v)
            tv = (bsum(sb_v[0]) + bsum(phs)) / nr
            ovec[...] = jnp.where(lanes == 0, tv, 0.0)
            pltpu.sync_copy(ovec, out.at[1])


def _k0_body(lpos_ref, lx_ref, rpos_ref, rx_ref, rb_ref, dt_ref,
             tlid_ref, trad_ref):
    pos = lpos_ref[...]
    x2 = lx_ref[:, 2:3]
    sq = jnp.sum(pos * pos, axis=1, keepdims=True)
    ones = jnp.ones_like(x2)
    z = jnp.zeros_like(pos[:, 0:2])
    tlid_ref[...] = jnp.concatenate([pos, x2, sq, ones, z], axis=1)

    rpos = rpos_ref[...]
    sp = jnp.abs(rx_ref[:, 2:3])
    b = rb_ref[...]
    nd = jnp.zeros_like(sp)
    for bb in range(8):
        nd = jnp.where(b == bb, dt_ref[0:1, bb:bb + 1], nd)
    nd = jnp.maximum(nd, 0.01)
    rz = jnp.zeros_like(rpos)
    trad_ref[...] = jnp.concatenate([rpos, sp, nd, rz], axis=1)


def _b_radar(a0, a1, p, ro, ndt):
    acc = a0 + a1
    s2 = acc[:, 0:3]
    s1 = acc[:, 4:5]
    cnt = acc[:, 5:6]
    rsq = jnp.sum(p * p, axis=1, keepdims=True)
    sum_d = cnt * rsq - 2.0 * jnp.sum(p * s2, axis=1, keepdims=True) + s1
    val = sum_d / jnp.maximum(cnt, 1.0) ** 2
    spat = jnp.where(cnt > 0, val, _GHOST)
    lv = jnp.clip(ro, _R_MIN, _R_MAX)
    den = 2.0 * jnp.exp(lv) * ndt * ndt + 1e-9
    return den, jnp.sum(spat / den + 0.5 * lv)


def _b_mind2(gs, gd, g, b2):
    ps = gs[:, 0:3]
    sp = gs[:, 3:4]
    nd = gs[:, 4:5]
    pd = gd[:, 0:3]
    mv = pd - ps
    nrm = jnp.sqrt(jnp.sum(mv * mv, axis=1, keepdims=True))
    unit = mv / (nrm + 1e-9)
    pred = ps + sp * unit * nd
    a2 = jnp.sum(pred * pred, axis=1, keepdims=True)
    d2 = jnp.maximum(
        a2 + b2 - 2.0 * jnp.dot(pred, g.T, preferred_element_type=jnp.float32),
        0.0)
    return jnp.min(d2, axis=1)


def _b_body(la0_ref, la1_ref, lpos_ref, lx_ref, lo_ref,
            r1a0_ref, r1a1_ref, r1pos_ref, r1o_ref, tr1_ref,
            r2a0_ref, r2a1_ref, r2pos_ref, r2o_ref, tr2_ref,
            g1s_ref, g1d_ref, g2s_ref, g2d_ref, gt_ref,
            md1_ref, md2_ref, den1_ref, den2_ref,
            sl_ref, s1_ref, s2_ref):
    i = pl.program_id(0)

    acc = la0_ref[0] + la1_ref[0]
    cnt = acc[:, 5:6]
    dc = jnp.maximum(cnt, 1.0)
    mp = acc[:, 0:3] / dc
    mi = acc[:, 3:4] / dc
    p = lpos_ref[...]
    res_pos = jnp.sum((p - mp) ** 2, axis=1, keepdims=True)
    res_int = (lx_ref[:, 2:3] - mi) ** 2
    lv = jnp.clip(lo_ref[...], _L_MIN, _L_MAX)
    prec = jnp.exp(-lv)
    tot_l = jnp.sum(0.5 * prec * res_pos + 0.5 * prec * res_int + 0.5 * lv)

    den1, tot_1 = _b_radar(r1a0_ref[0], r1a1_ref[0], r1pos_ref[...],
                           r1o_ref[...], tr1_ref[:, 4:5])
    den2, tot_2 = _b_radar(r2a0_ref[0], r2a1_ref[0], r2pos_ref[...],
                           r2o_ref[...], tr2_ref[:, 4:5])
    den1_ref[...] = den1
    den2_ref[...] = den2

    g = gt_ref[...]
    b2 = jnp.sum(g * g, axis=1)[None, :]
    md1_ref[...] = _b_mind2(g1s_ref[...], g1d_ref[...], g, b2)
    md2_ref[...] = _b_mind2(g2s_ref[...], g2d_ref[...], g, b2)

    iota16 = lax.broadcasted_iota(jnp.int32, (1, 16), 1)
    @pl.when(i == 0)
    def _():
        sl_ref[...] = jnp.zeros((1, 16), jnp.float32)
        s1_ref[...] = jnp.zeros((1, 16), jnp.float32)
        s2_ref[...] = jnp.zeros((1, 16), jnp.float32)
    sl_ref[...] += jnp.where(iota16 == 0, tot_l, 0.0)
    s1_ref[...] += jnp.where(iota16 == 0, tot_1, 0.0)
    s2_ref[...] += jnp.where(iota16 == 0, tot_2, 0.0)


def _blk(shape, imap):
    return pl.BlockSpec(shape, imap)


def kernel(lidar_out, lidar_pos, lidar_x, lidar_spatial_edge_index, radar1_out,
           radar1_pos, radar1_x, radar1_batch, radar1_temporal_edge_index,
           radar1_to_lidar_src, radar1_to_lidar_dst, radar2_out, radar2_pos,
           radar2_x, radar2_batch, radar2_temporal_edge_index,
           radar2_to_lidar_src, radar2_to_lidar_dst, dt_sec, gt_radar_pos):
    f32, i32 = jnp.float32, jnp.int32

    lid_ei = lidar_spatial_edge_index.astype(i32)
    pad_c = _ECP - 200000
    pad_t = _ETP - 100000
    cpad_s = _NR + (jnp.arange(pad_c, dtype=i32) % (_NRP - _NR))
    cpad_d = jnp.arange(pad_c, dtype=i32) % _NL
    r1cs = jnp.concatenate([radar1_to_lidar_src.astype(i32), cpad_s])
    r1cd = jnp.concatenate([radar1_to_lidar_dst.astype(i32), cpad_d])
    r2cs = jnp.concatenate([radar2_to_lidar_src.astype(i32), cpad_s])
    r2cd = jnp.concatenate([radar2_to_lidar_dst.astype(i32), cpad_d])
    tpad = jnp.broadcast_to(jnp.arange(pad_t, dtype=i32) % _NR, (2, pad_t))
    tei1 = jnp.concatenate([radar1_temporal_edge_index.astype(i32), tpad], axis=1)
    tei2 = jnp.concatenate([radar2_temporal_edge_index.astype(i32), tpad], axis=1)
    zeros = jnp.zeros((_NL // 16, _D), f32)
    dt2 = dt_sec.reshape(1, 8).astype(f32)

    rpos = jnp.concatenate([radar1_pos, radar2_pos], axis=0)
    rx = jnp.concatenate([radar1_x, radar2_x], axis=0)
    rb = jnp.concatenate([radar1_batch.astype(i32),
                          radar2_batch.astype(i32)]).reshape(2 * _NR, 1)
    t_lid, t_rad = pl.pallas_call(
        _k0_body,
        grid=(100,),
        in_specs=[_blk((1000, 3), lambda i: (i, 0)),
                  _blk((1000, 3), lambda i: (i, 0)),
                  _blk((400, 3), lambda i: (i, 0)),
                  _blk((400, 3), lambda i: (i, 0)),
                  _blk((400, 1), lambda i: (i, 0)),
                  _blk((1, 8), lambda i: (0, 0))],
        out_specs=[_blk((1000, _D), lambda i: (i, 0)),
                   _blk((400, _D), lambda i: (i, 0))],
        out_shape=[jax.ShapeDtypeStruct((_NL, _D), f32),
                   jax.ShapeDtypeStruct((2 * _NR, _D), f32)],
    )(lidar_pos, lidar_x, rpos, rx, rb, dt2)
    t_r1 = t_rad[:_NR]
    t_r2 = t_rad[_NR:]

    lid_acc, r1_acc, r2_acc, g1s, g1d, g2s, g2d = _sc_edge_pass(
        t_lid, t_r1, t_r2, lid_ei, r1cs, r1cd, r2cs, r2cd, tei1, tei2, zeros)

    LB, RB, TB = 800, 160, 1024
    (md1, md2, den1, den2, s_lid, s_r1, s_r2) = pl.pallas_call(
        _b_body,
        grid=(125,),
        in_specs=[_blk((1, LB, _D), lambda i: (0, i, 0)),
                  _blk((1, LB, _D), lambda i: (1, i, 0)),
                  _blk((LB, 3), lambda i: (i, 0)),
                  _blk((LB, 3), lambda i: (i, 0)),
                  _blk((LB, 1), lambda i: (i, 0)),
                  _blk((1, RB, _D), lambda i: (0, i, 0)),
                  _blk((1, RB, _D), lambda i: (1, i, 0)),
                  _blk((RB, 3), lambda i: (i, 0)),
                  _blk((RB, 1), lambda i: (i, 0)),
                  _blk((RB, _D), lambda i: (i, 0)),
                  _blk((1, RB, _D), lambda i: (0, i, 0)),
                  _blk((1, RB, _D), lambda i: (1, i, 0)),
                  _blk((RB, 3), lambda i: (i, 0)),
                  _blk((RB, 1), lambda i: (i, 0)),
                  _blk((RB, _D), lambda i: (i, 0)),
                  _blk((TB, _D), lambda i: (i, 0)),
                  _blk((TB, _D), lambda i: (i, 0)),
                  _blk((TB, _D), lambda i: (i, 0)),
                  _blk((TB, _D), lambda i: (i, 0)),
                  _blk((_NGT, 3), lambda i: (0, 0))],
        out_specs=[_blk((TB,), lambda i: (i,)),
                   _blk((TB,), lambda i: (i,)),
                   _blk((RB, 1), lambda i: (i, 0)),
                   _blk((RB, 1), lambda i: (i, 0)),
                   _blk((1, 16), lambda i: (0, 0)),
                   _blk((1, 16), lambda i: (0, 0)),
                   _blk((1, 16), lambda i: (0, 0))],
        out_shape=[jax.ShapeDtypeStruct((_ETP,), f32),
                   jax.ShapeDtypeStruct((_ETP,), f32),
                   jax.ShapeDtypeStruct((_NR, 1), f32),
                   jax.ShapeDtypeStruct((_NR, 1), f32),
                   jax.ShapeDtypeStruct((1, 16), f32),
                   jax.ShapeDtypeStruct((1, 16), f32),
                   jax.ShapeDtypeStruct((1, 16), f32)],
    )(lid_acc, lid_acc, lidar_pos, lidar_x, lidar_out,
      r1_acc, r1_acc, radar1_pos, radar1_out, t_r1,
      r2_acc, r2_acc, radar2_pos, radar2_out, t_r2,
      g1s, g1d, g2s, g2d, gt_radar_pos)

    tei_b = jnp.stack([tei1, tei2])
    md_b = jnp.stack([md1, md2])
    den_b = jnp.stack([den1.reshape(_NR), den2.reshape(_NR)])
    tot = _sc_phys_pass(tei_b, md_b, den_b, s_lid, s_r1, s_r2)
    return tot[0, 0] + tot[1, 0]


# speedup vs baseline: 1.6846x; 1.1628x over previous
import functools
import math

import jax
import jax.numpy as jnp
from jax import lax
from jax.experimental import pallas as pl
from jax.experimental.pallas import tpu as pltpu
from jax.experimental.pallas import tpu_sc as plsc

_SCALE_POSE = 10.0
_SCALE_RADAR_V = 5.0
_L_MIN = 2 * math.log(0.03 / _SCALE_POSE + 1e-09)
_L_MAX = 2 * math.log(0.5 / _SCALE_POSE + 1e-09)
_R_MIN = 2 * math.log(0.1 / _SCALE_RADAR_V + 1e-09)
_R_MAX = 2 * math.log(5.0 / _SCALE_RADAR_V + 1e-09)
_GHOST = (0.6 / _SCALE_POSE) ** 2

_NL = 100000
_EL = 1600000
_NR = 20000
_NRP = 20480
_ETP = 128000
_ECP = 256000
_NGT = 256
_D = 8
_CH = 2000

_mesh = plsc.VectorSubcoreMesh(core_axis_name="c", subcore_axis_name="s")
_sc_params = pltpu.CompilerParams(use_tc_tiling_on_sc=False)
_sc_params_nl = pltpu.CompilerParams(
    use_tc_tiling_on_sc=False, needs_layout_passes=False)


def _pipelined_pass(n, base0, ei0, ei1, table, sink, src_v, dst_v, rows_v,
                    sem_i, sem_g, sem_s, write_linear=False, gout=None):
    def idx_copies(k):
        b = k % 2
        ops = [(ei0.at[0, pl.ds(base0 + k * _CH, _CH)] if ei0.ndim == 2
                else ei0.at[pl.ds(base0 + k * _CH, _CH)], src_v[b], sem_i[b])]
        if not write_linear:
            ops.append((ei1.at[1, pl.ds(base0 + k * _CH, _CH)] if ei1.ndim == 2
                        else ei1.at[pl.ds(base0 + k * _CH, _CH)],
                        dst_v[b], sem_i[b]))
        return ops

    def start_idx(k):
        for s_, d_, m_ in idx_copies(k):
            pltpu.async_copy(s_, d_, m_)

    def wait_idx(k):
        for s_, d_, m_ in idx_copies(k):
            pltpu.make_async_copy(s_, d_, m_).wait()

    def gather_args(k):
        b = k % 2
        return table.at[src_v[b]], rows_v[b], sem_g[b]

    def sink_args(k):
        b = k % 2
        if write_linear:
            return rows_v[b], gout.at[pl.ds(base0 + k * _CH, _CH)], sem_s[b]
        return rows_v[b], sink.at[dst_v[b]], sem_s[b]

    start_idx(0)
    for k in range(n):
        wait_idx(k)
        pltpu.async_copy(*gather_args(k))
        if k + 1 < n:
            if k >= 1:
                pltpu.make_async_copy(*sink_args(k - 1)).wait()
            start_idx(k + 1)
        elif k >= 1:
            pltpu.make_async_copy(*sink_args(k - 1)).wait()
        pltpu.make_async_copy(*gather_args(k)).wait()
        if write_linear:
            pltpu.async_copy(*sink_args(k))
        else:
            s_, d_, m_ = sink_args(k)
            pltpu.async_copy(s_, d_, m_, add=True)
    pltpu.make_async_copy(*sink_args(n - 1)).wait()


@functools.partial(
    pl.kernel,
    out_type=(
        jax.ShapeDtypeStruct((2, _NL, _D), jnp.float32),
        jax.ShapeDtypeStruct((2, _NRP, _D), jnp.float32),
        jax.ShapeDtypeStruct((2, _NRP, _D), jnp.float32),
        jax.ShapeDtypeStruct((_ETP, _D), jnp.float32),
        jax.ShapeDtypeStruct((_ETP, _D), jnp.float32),
        jax.ShapeDtypeStruct((_ETP, _D), jnp.float32),
        jax.ShapeDtypeStruct((_ETP, _D), jnp.float32),
    ),
    scratch_types=[
        pltpu.VMEM((_CH,), jnp.int32),
        pltpu.VMEM((_CH,), jnp.int32),
        pltpu.VMEM((_CH,), jnp.int32),
        pltpu.VMEM((_CH,), jnp.int32),
        pltpu.VMEM((_CH, _D), jnp.float32),
        pltpu.VMEM((_CH, _D), jnp.float32),
        pltpu.VMEM_SHARED((_NL, _D), jnp.float32),
        pltpu.VMEM_SHARED((_NRP, _D), jnp.float32),
        pltpu.VMEM_SHARED((_NRP, _D), jnp.float32),
        pltpu.SemaphoreType.DMA,
        pltpu.SemaphoreType.DMA,
        pltpu.SemaphoreType.DMA,
        pltpu.SemaphoreType.DMA,
        pltpu.SemaphoreType.DMA,
        pltpu.SemaphoreType.DMA,
    ],
    mesh=_mesh,
    compiler_params=_sc_params,
)
def _sc_edge_pass(t_lid, t_r1, t_r2, lid_ei, r1cs, r1cd, r2cs, r2cd, tei1,
                  tei2, zeros, lid_acc, r1_acc, r2_acc, g1s, g1d, g2s, g2d,
                  src_v0, src_v1, dst_v0, dst_v1, rows_v0, rows_v1,
                  accl, acc1, acc2, semi0, semi1, semg0, semg1, sems0, sems1):
    c = lax.axis_index("c")
    s = lax.axis_index("s")
    wid = c * 16 + s
    src_v = (src_v0, src_v1)
    dst_v = (dst_v0, dst_v1)
    rows_v = (rows_v0, rows_v1)
    sem_i = (semi0, semi1)
    sem_g = (semg0, semg1)
    sem_s = (sems0, sems1)

    nl16 = _NL // 16
    nr16 = _NRP // 16
    pltpu.sync_copy(zeros, accl.at[pl.ds(s * nl16, nl16)])
    pltpu.sync_copy(zeros.at[pl.ds(0, nr16)], acc1.at[pl.ds(s * nr16, nr16)])
    pltpu.sync_copy(zeros.at[pl.ds(0, nr16)], acc2.at[pl.ds(s * nr16, nr16)])
    plsc.subcore_barrier()

    common = dict(src_v=src_v, dst_v=dst_v, rows_v=rows_v,
                  sem_i=sem_i, sem_g=sem_g, sem_s=sem_s)
    _pipelined_pass(_EL // 32 // _CH, wid * (_EL // 32), lid_ei, lid_ei,
                    t_lid, accl, **common)
    for cs_ref, cd_ref, acc in ((r1cs, r1cd, acc1), (r2cs, r2cd, acc2)):
        _pipelined_pass(_ECP // 32 // _CH, wid * (_ECP // 32), cd_ref, cs_ref,
                        t_lid, acc, **common)
    for tei, t_r, gs, gd in ((tei1, t_r1, g1s, g1d), (tei2, t_r2, g2s, g2d)):
        _pipelined_pass(_ETP // 32 // _CH, wid * (_ETP // 32), tei, None,
                        t_r, None, write_linear=True, gout=gs, **common)
        tei_dst = tei.at[1]
        _pipelined_pass(_ETP // 32 // _CH, wid * (_ETP // 32), tei_dst, None,
                        t_r, None, write_linear=True, gout=gd, **common)

    plsc.subcore_barrier()
    pltpu.sync_copy(accl.at[pl.ds(s * nl16, nl16)], lid_acc.at[c, pl.ds(s * nl16, nl16)])
    pltpu.sync_copy(acc1.at[pl.ds(s * nr16, nr16)], r1_acc.at[c, pl.ds(s * nr16, nr16)])
    pltpu.sync_copy(acc2.at[pl.ds(s * nr16, nr16)], r2_acc.at[c, pl.ds(s * nr16, nr16)])


_NEC = 50
_NRE = 20224
_MS = 1264

@functools.partial(
    pl.kernel,
    out_type=jax.ShapeDtypeStruct((2, 16), jnp.float32),
    scratch_types=[
        pltpu.VMEM((_CH,), jnp.int32),
        pltpu.VMEM((_CH,), jnp.float32),
        pltpu.VMEM((_NRE,), jnp.int32),
        pltpu.VMEM((_MS,), jnp.int32),
        pltpu.VMEM((_NR,), jnp.float32),
        pltpu.VMEM((16,), jnp.float32),
        pltpu.VMEM((16, 16), jnp.float32),
        pltpu.VMEM((1, 16), jnp.float32),
        pltpu.VMEM((1, 16), jnp.float32),
        pltpu.VMEM_SHARED((16, _NRE), jnp.int32),
        pltpu.VMEM_SHARED((_NRE,), jnp.int32),
        pltpu.VMEM_SHARED((16, 16), jnp.float32),
    ],
    mesh=_mesh,
    compiler_params=_sc_params_nl,
)
def _sc_phys_pass(tei_b, md_b, den_b, sl, sr1, sr2, out,
                  src_v, md_v, eid_v, tmp_v, den_v, ovec, ph_v, sa_v, sb_v,
                  eid_sp, merged_sp, phys_sp):
    c = lax.axis_index("c")
    s = lax.axis_index("s")
    lanes = lax.iota(jnp.int32, 16)

    zi = jnp.zeros((16,), jnp.int32)
    def init_body(i, _):
        eid_v[pl.ds(i * 16, 16)] = zi
        return 0
    lax.fori_loop(0, _NRE // 16, init_body, 0)

    def p1_chunk(k):
        pltpu.sync_copy(tei_b.at[c, 0, pl.ds(k * _CH, _CH)], src_v)
        def p1_vreg(j, _):
            idx = src_v[pl.ds(j * 16, 16)]
            my = (k * _CH + j * 16 + 1) + lanes
            plsc.store_scatter(eid_v, [idx], my)
            def rnd(r, _):
                g = plsc.load_gather(eid_v, [idx])
                m = my > g
                @pl.when(jnp.any(m))
                def _():
                    plsc.store_scatter(eid_v, [idx], my, mask=m)
                return 0
            lax.fori_loop(0, 3, rnd, 0)
            return 0
        lax.fori_loop(0, _CH // 16, p1_vreg, 0)

    for r in range((_NEC + 15) // 16):
        k = s + 16 * r
        @pl.when(k < _NEC)
        def _(k=k):
            p1_chunk(k)

    pltpu.sync_copy(eid_v, eid_sp.at[s])
    plsc.subcore_barrier()
    pltpu.sync_copy(eid_sp.at[0, pl.ds(s * _MS, _MS)], tmp_v)
    def cp_body(j, _):
        eid_v[pl.ds(j * 16, 16)] = tmp_v[pl.ds(j * 16, 16)]
        return 0
    lax.fori_loop(0, _MS // 16, cp_body, 0)
    for rr in range(1, 16):
        pltpu.sync_copy(eid_sp.at[rr, pl.ds(s * _MS, _MS)], tmp_v)
        def mx_body(j, _):
            eid_v[pl.ds(j * 16, 16)] = jnp.maximum(
                eid_v[pl.ds(j * 16, 16)], tmp_v[pl.ds(j * 16, 16)])
            return 0
        lax.fori_loop(0, _MS // 16, mx_body, 0)
    pltpu.sync_copy(eid_v.at[pl.ds(0, _MS)], merged_sp.at[pl.ds(s * _MS, _MS)])
    plsc.subcore_barrier()
    pltpu.sync_copy(merged_sp, eid_v)
    pltpu.sync_copy(den_b.at[c], den_v)

    ovec[...] = jnp.zeros((16,), jnp.float32)
    def p2_chunk(k):
        pltpu.sync_copy(tei_b.at[c, 0, pl.ds(k * _CH, _CH)], src_v)
        pltpu.sync_copy(md_b.at[c, pl.ds(k * _CH, _CH)], md_v)
        def p2_vreg(j, acc):
            idx = src_v[pl.ds(j * 16, 16)]
            my = (k * _CH + j * 16 + 1) + lanes
            g = plsc.load_gather(eid_v, [idx])
            w = g == my
            dg = plsc.load_gather(den_v, [idx])
            mdv = md_v[pl.ds(j * 16, 16)]
            return acc + jnp.where(w, mdv / dg, 0.0)
        acc = lax.fori_loop(0, _CH // 16, p2_vreg, jnp.zeros((16,), jnp.float32))
        ovec[...] += acc

    for r in range((_NEC + 15) // 16):
        k = s + 16 * r
        @pl.when(k < _NEC)
        def _(k=k):
            p2_chunk(k)

    pltpu.sync_copy(ovec, phys_sp.at[s])
    plsc.subcore_barrier()

    @pl.when(s == 0)
    def _():
        pltpu.sync_copy(phys_sp, ph_v)
        phs = jnp.zeros((16,), jnp.float32)
        for rr in range(16):
            phs = phs + ph_v[rr]
        def bsum(v):
            return jnp.broadcast_to(jnp.sum(v), (16,))
        nl = jnp.full((16,), float(_NL), jnp.float32)
        nr = jnp.full((16,), float(_NR), jnp.float32)

        @pl.when(c == 0)
        def _():
            pltpu.sync_copy(sl, sa_v)
            pltpu.sync_copy(sr1, sb_v)
            tv = bsum(sa_v[0]) / nl + (bsum(sb_v[0]) + bsum(phs)) / nr
            ovec[...] = jnp.where(lanes == 0, tv, 0.0)
            pltpu.sync_copy(ovec, out.at[0])

        @pl.when(c == 1)
        def _():
            pltpu.sync_copy(sr2, sb_v)
            tv = (bsum(sb_v[0]) + bsum(phs)) / nr
            ovec[...] = jnp.where(lanes == 0, tv, 0.0)
            pltpu.sync_copy(ovec, out.at[1])


def _k0_body(lpos_ref, lx_ref, rpos_ref, rx_ref, rb_ref, dt_ref,
             tlid_ref, trad_ref):
    pos = lpos_ref[...]
    x2 = lx_ref[:, 2:3]
    sq = jnp.sum(pos * pos, axis=1, keepdims=True)
    ones = jnp.ones_like(x2)
    z = jnp.zeros_like(pos[:, 0:2])
    tlid_ref[...] = jnp.concatenate([pos, x2, sq, ones, z], axis=1)

    rpos = rpos_ref[...]
    sp = jnp.abs(rx_ref[:, 2:3])
    b = rb_ref[...]
    nd = jnp.zeros_like(sp)
    for bb in range(8):
        nd = jnp.where(b == bb, dt_ref[0:1, bb:bb + 1], nd)
    nd = jnp.maximum(nd, 0.01)
    rz = jnp.zeros_like(rpos)
    trad_ref[...] = jnp.concatenate([rpos, sp, nd, rz], axis=1)


def _b_radar(a0, a1, p, ro, ndt):
    acc = jnp.transpose(a0 + a1)
    s2 = acc[0:3]
    s1 = acc[4:5]
    cnt = acc[5:6]
    rsq = jnp.sum(p * p, axis=0, keepdims=True)
    sum_d = cnt * rsq - 2.0 * jnp.sum(p * s2, axis=0, keepdims=True) + s1
    val = sum_d / jnp.maximum(cnt, 1.0) ** 2
    spat = jnp.where(cnt > 0, val, _GHOST)
    lv = jnp.clip(ro, _R_MIN, _R_MAX)
    den = 2.0 * jnp.exp(lv) * ndt * ndt + 1e-9
    return den, jnp.sum(spat / den + 0.5 * lv)


def _b_mind2(gs, gd, g, b2):
    gst = jnp.transpose(gs)
    gdt = jnp.transpose(gd)
    ps = gst[0:3]
    sp = gst[3:4]
    nd = gst[4:5]
    pd = gdt[0:3]
    mv = pd - ps
    nrm = jnp.sqrt(jnp.sum(mv * mv, axis=0, keepdims=True))
    unit = mv / (nrm + 1e-9)
    pred = ps + sp * unit * nd
    a2 = jnp.sum(pred * pred, axis=0, keepdims=True)
    d2 = jnp.maximum(
        a2 + b2 - 2.0 * jnp.dot(g, pred, preferred_element_type=jnp.float32),
        0.0)
    return jnp.min(d2, axis=0)


def _b_body(la0_ref, la1_ref, lpos_ref, lx_ref, lo_ref,
            r1a0_ref, r1a1_ref, r1pos_ref, r1o_ref, tr1_ref,
            r2a0_ref, r2a1_ref, r2pos_ref, r2o_ref, tr2_ref,
            g1s_ref, g1d_ref, g2s_ref, g2d_ref, gt_ref,
            md1_ref, md2_ref, den1_ref, den2_ref,
            sl_ref, s1_ref, s2_ref):
    i = pl.program_id(0)

    acc = jnp.transpose(la0_ref[0] + la1_ref[0])
    cnt = acc[5:6]
    dc = jnp.maximum(cnt, 1.0)
    mp = acc[0:3] / dc
    mi = acc[3:4] / dc
    p = jnp.transpose(lpos_ref[...])
    res_pos = jnp.sum((p - mp) ** 2, axis=0, keepdims=True)
    res_int = (jnp.transpose(lx_ref[...])[2:3] - mi) ** 2
    lv = jnp.clip(jnp.transpose(lo_ref[...]), _L_MIN, _L_MAX)
    prec = jnp.exp(-lv)
    tot_l = jnp.sum(0.5 * prec * res_pos + 0.5 * prec * res_int + 0.5 * lv)

    den1, tot_1 = _b_radar(r1a0_ref[0], r1a1_ref[0],
                           jnp.transpose(r1pos_ref[...]),
                           jnp.transpose(r1o_ref[...]),
                           jnp.transpose(tr1_ref[...])[4:5])
    den2, tot_2 = _b_radar(r2a0_ref[0], r2a1_ref[0],
                           jnp.transpose(r2pos_ref[...]),
                           jnp.transpose(r2o_ref[...]),
                           jnp.transpose(tr2_ref[...])[4:5])
    den1_ref[...] = jnp.transpose(den1)
    den2_ref[...] = jnp.transpose(den2)

    g = gt_ref[...]
    b2 = jnp.sum(g * g, axis=1, keepdims=True)
    md1_ref[...] = _b_mind2(g1s_ref[...], g1d_ref[...], g, b2)
    md2_ref[...] = _b_mind2(g2s_ref[...], g2d_ref[...], g, b2)

    iota16 = lax.broadcasted_iota(jnp.int32, (1, 16), 1)
    @pl.when(i == 0)
    def _():
        sl_ref[...] = jnp.zeros((1, 16), jnp.float32)
        s1_ref[...] = jnp.zeros((1, 16), jnp.float32)
        s2_ref[...] = jnp.zeros((1, 16), jnp.float32)
    sl_ref[...] += jnp.where(iota16 == 0, tot_l, 0.0)
    s1_ref[...] += jnp.where(iota16 == 0, tot_1, 0.0)
    s2_ref[...] += jnp.where(iota16 == 0, tot_2, 0.0)


def _blk(shape, imap):
    return pl.BlockSpec(shape, imap)


def kernel(lidar_out, lidar_pos, lidar_x, lidar_spatial_edge_index, radar1_out,
           radar1_pos, radar1_x, radar1_batch, radar1_temporal_edge_index,
           radar1_to_lidar_src, radar1_to_lidar_dst, radar2_out, radar2_pos,
           radar2_x, radar2_batch, radar2_temporal_edge_index,
           radar2_to_lidar_src, radar2_to_lidar_dst, dt_sec, gt_radar_pos):
    f32, i32 = jnp.float32, jnp.int32

    lid_ei = lidar_spatial_edge_index.astype(i32)
    pad_c = _ECP - 200000
    pad_t = _ETP - 100000
    cpad_s = _NR + (jnp.arange(pad_c, dtype=i32) % (_NRP - _NR))
    cpad_d = jnp.arange(pad_c, dtype=i32) % _NL
    r1cs = jnp.concatenate([radar1_to_lidar_src.astype(i32), cpad_s])
    r1cd = jnp.concatenate([radar1_to_lidar_dst.astype(i32), cpad_d])
    r2cs = jnp.concatenate([radar2_to_lidar_src.astype(i32), cpad_s])
    r2cd = jnp.concatenate([radar2_to_lidar_dst.astype(i32), cpad_d])
    tpad = jnp.broadcast_to(jnp.arange(pad_t, dtype=i32) % _NR, (2, pad_t))
    tei1 = jnp.concatenate([radar1_temporal_edge_index.astype(i32), tpad], axis=1)
    tei2 = jnp.concatenate([radar2_temporal_edge_index.astype(i32), tpad], axis=1)
    zeros = jnp.zeros((_NL // 16, _D), f32)
    dt2 = dt_sec.reshape(1, 8).astype(f32)

    rpos = jnp.concatenate([radar1_pos, radar2_pos], axis=0)
    rx = jnp.concatenate([radar1_x, radar2_x], axis=0)
    rb = jnp.concatenate([radar1_batch.astype(i32),
                          radar2_batch.astype(i32)]).reshape(2 * _NR, 1)
    t_lid, t_rad = pl.pallas_call(
        _k0_body,
        grid=(100,),
        in_specs=[_blk((1000, 3), lambda i: (i, 0)),
                  _blk((1000, 3), lambda i: (i, 0)),
                  _blk((400, 3), lambda i: (i, 0)),
                  _blk((400, 3), lambda i: (i, 0)),
                  _blk((400, 1), lambda i: (i, 0)),
                  _blk((1, 8), lambda i: (0, 0))],
        out_specs=[_blk((1000, _D), lambda i: (i, 0)),
                   _blk((400, _D), lambda i: (i, 0))],
        out_shape=[jax.ShapeDtypeStruct((_NL, _D), f32),
                   jax.ShapeDtypeStruct((2 * _NR, _D), f32)],
    )(lidar_pos, lidar_x, rpos, rx, rb, dt2)
    t_r1 = t_rad[:_NR]
    t_r2 = t_rad[_NR:]

    lid_acc, r1_acc, r2_acc, g1s, g1d, g2s, g2d = _sc_edge_pass(
        t_lid, t_r1, t_r2, lid_ei, r1cs, r1cd, r2cs, r2cd, tei1, tei2, zeros)

    LB, RB, TB = 800, 160, 1024
    (md1, md2, den1, den2, s_lid, s_r1, s_r2) = pl.pallas_call(
        _b_body,
        grid=(125,),
        in_specs=[_blk((1, LB, _D), lambda i: (0, i, 0)),
                  _blk((1, LB, _D), lambda i: (1, i, 0)),
                  _blk((LB, 3), lambda i: (i, 0)),
                  _blk((LB, 3), lambda i: (i, 0)),
                  _blk((LB, 1), lambda i: (i, 0)),
                  _blk((1, RB, _D), lambda i: (0, i, 0)),
                  _blk((1, RB, _D), lambda i: (1, i, 0)),
                  _blk((RB, 3), lambda i: (i, 0)),
                  _blk((RB, 1), lambda i: (i, 0)),
                  _blk((RB, _D), lambda i: (i, 0)),
                  _blk((1, RB, _D), lambda i: (0, i, 0)),
                  _blk((1, RB, _D), lambda i: (1, i, 0)),
                  _blk((RB, 3), lambda i: (i, 0)),
                  _blk((RB, 1), lambda i: (i, 0)),
                  _blk((RB, _D), lambda i: (i, 0)),
                  _blk((TB, _D), lambda i: (i, 0)),
                  _blk((TB, _D), lambda i: (i, 0)),
                  _blk((TB, _D), lambda i: (i, 0)),
                  _blk((TB, _D), lambda i: (i, 0)),
                  _blk((_NGT, 3), lambda i: (0, 0))],
        out_specs=[_blk((TB,), lambda i: (i,)),
                   _blk((TB,), lambda i: (i,)),
                   _blk((RB, 1), lambda i: (i, 0)),
                   _blk((RB, 1), lambda i: (i, 0)),
                   _blk((1, 16), lambda i: (0, 0)),
                   _blk((1, 16), lambda i: (0, 0)),
                   _blk((1, 16), lambda i: (0, 0))],
        out_shape=[jax.ShapeDtypeStruct((_ETP,), f32),
                   jax.ShapeDtypeStruct((_ETP,), f32),
                   jax.ShapeDtypeStruct((_NR, 1), f32),
                   jax.ShapeDtypeStruct((_NR, 1), f32),
                   jax.ShapeDtypeStruct((1, 16), f32),
                   jax.ShapeDtypeStruct((1, 16), f32),
                   jax.ShapeDtypeStruct((1, 16), f32)],
    )(lid_acc, lid_acc, lidar_pos, lidar_x, lidar_out,
      r1_acc, r1_acc, radar1_pos, radar1_out, t_r1,
      r2_acc, r2_acc, radar2_pos, radar2_out, t_r2,
      g1s, g1d, g2s, g2d, gt_radar_pos)

    tei_b = jnp.stack([tei1, tei2])
    md_b = jnp.stack([md1, md2])
    den_b = jnp.stack([den1.reshape(_NR), den2.reshape(_NR)])
    tot = _sc_phys_pass(tei_b, md_b, den_b, s_lid, s_r1, s_r2)
    return tot[0, 0] + tot[1, 0]


# speedup vs baseline: 1.7311x; 1.0276x over previous
import functools
import math

import jax
import jax.numpy as jnp
from jax import lax
from jax.experimental import pallas as pl
from jax.experimental.pallas import tpu as pltpu
from jax.experimental.pallas import tpu_sc as plsc

_SCALE_POSE = 10.0
_SCALE_RADAR_V = 5.0
_L_MIN = 2 * math.log(0.03 / _SCALE_POSE + 1e-09)
_L_MAX = 2 * math.log(0.5 / _SCALE_POSE + 1e-09)
_R_MIN = 2 * math.log(0.1 / _SCALE_RADAR_V + 1e-09)
_R_MAX = 2 * math.log(5.0 / _SCALE_RADAR_V + 1e-09)
_GHOST = (0.6 / _SCALE_POSE) ** 2

_NL = 100000
_EL = 1600000
_NR = 20000
_NRP = 20480
_ETP = 128000
_ECP = 256000
_NGT = 256
_D = 8
_CH = 2000

_mesh = plsc.VectorSubcoreMesh(core_axis_name="c", subcore_axis_name="s")
_sc_params = pltpu.CompilerParams(use_tc_tiling_on_sc=False)
_sc_params_nl = pltpu.CompilerParams(
    use_tc_tiling_on_sc=False, needs_layout_passes=False)


def _pipelined_pass(n, base0, ei0, ei1, table, sink, src_v, dst_v, rows_v,
                    sem_i, sem_g, sem_s, write_linear=False, gout=None):
    def idx_copies(k):
        b = k % 2
        ops = [(ei0.at[0, pl.ds(base0 + k * _CH, _CH)] if ei0.ndim == 2
                else ei0.at[pl.ds(base0 + k * _CH, _CH)], src_v[b], sem_i[b])]
        if not write_linear:
            ops.append((ei1.at[1, pl.ds(base0 + k * _CH, _CH)] if ei1.ndim == 2
                        else ei1.at[pl.ds(base0 + k * _CH, _CH)],
                        dst_v[b], sem_i[b]))
        return ops

    def start_idx(k):
        for s_, d_, m_ in idx_copies(k):
            pltpu.async_copy(s_, d_, m_)

    def wait_idx(k):
        for s_, d_, m_ in idx_copies(k):
            pltpu.make_async_copy(s_, d_, m_).wait()

    def gather_args(k):
        b = k % 2
        return table.at[src_v[b]], rows_v[b], sem_g[b]

    def sink_args(k):
        b = k % 2
        if write_linear:
            return rows_v[b], gout.at[pl.ds(base0 + k * _CH, _CH)], sem_s[b]
        return rows_v[b], sink.at[dst_v[b]], sem_s[b]

    start_idx(0)
    for k in range(n):
        wait_idx(k)
        pltpu.async_copy(*gather_args(k))
        if k + 1 < n:
            if k >= 1:
                pltpu.make_async_copy(*sink_args(k - 1)).wait()
            start_idx(k + 1)
        elif k >= 1:
            pltpu.make_async_copy(*sink_args(k - 1)).wait()
        pltpu.make_async_copy(*gather_args(k)).wait()
        if write_linear:
            pltpu.async_copy(*sink_args(k))
        else:
            s_, d_, m_ = sink_args(k)
            pltpu.async_copy(s_, d_, m_, add=True)
    pltpu.make_async_copy(*sink_args(n - 1)).wait()


_sc_scratch = [
    pltpu.VMEM((_CH,), jnp.int32),
    pltpu.VMEM((_CH,), jnp.int32),
    pltpu.VMEM((_CH,), jnp.int32),
    pltpu.VMEM((_CH,), jnp.int32),
    pltpu.VMEM((_CH, _D), jnp.float32),
    pltpu.VMEM((_CH, _D), jnp.float32),
    pltpu.SemaphoreType.DMA,
    pltpu.SemaphoreType.DMA,
    pltpu.SemaphoreType.DMA,
    pltpu.SemaphoreType.DMA,
    pltpu.SemaphoreType.DMA,
    pltpu.SemaphoreType.DMA,
]


@functools.partial(
    pl.kernel,
    out_type=(
        jax.ShapeDtypeStruct((_ETP, _D), jnp.float32),
        jax.ShapeDtypeStruct((_ETP, _D), jnp.float32),
        jax.ShapeDtypeStruct((_ETP, _D), jnp.float32),
        jax.ShapeDtypeStruct((_ETP, _D), jnp.float32),
    ),
    scratch_types=_sc_scratch,
    mesh=_mesh,
    compiler_params=_sc_params,
)
def _sc_temporal_pass(t_r1, t_r2, tei1, tei2, g1s, g1d, g2s, g2d,
                      src_v0, src_v1, dst_v0, dst_v1, rows_v0, rows_v1,
                      semi0, semi1, semg0, semg1, sems0, sems1):
    c = lax.axis_index("c")
    s = lax.axis_index("s")
    wid = c * 16 + s
    common = dict(src_v=(src_v0, src_v1), dst_v=(dst_v0, dst_v1),
                  rows_v=(rows_v0, rows_v1), sem_i=(semi0, semi1),
                  sem_g=(semg0, semg1), sem_s=(sems0, sems1))
    for tei, t_r, gs, gd in ((tei1, t_r1, g1s, g1d), (tei2, t_r2, g2s, g2d)):
        _pipelined_pass(_ETP // 32 // _CH, wid * (_ETP // 32), tei, None,
                        t_r, None, write_linear=True, gout=gs, **common)
        tei_dst = tei.at[1]
        _pipelined_pass(_ETP // 32 // _CH, wid * (_ETP // 32), tei_dst, None,
                        t_r, None, write_linear=True, gout=gd, **common)


@functools.partial(
    pl.kernel,
    out_type=(
        jax.ShapeDtypeStruct((2, _NL, _D), jnp.float32),
        jax.ShapeDtypeStruct((2, _NRP, _D), jnp.float32),
        jax.ShapeDtypeStruct((2, _NRP, _D), jnp.float32),
    ),
    scratch_types=_sc_scratch + [
        pltpu.VMEM_SHARED((_NL, _D), jnp.float32),
        pltpu.VMEM_SHARED((_NRP, _D), jnp.float32),
        pltpu.VMEM_SHARED((_NRP, _D), jnp.float32),
    ],
    mesh=_mesh,
    compiler_params=_sc_params,
)
def _sc_edge_pass(t_lid, lid_ei, r1cs, r1cd, r2cs, r2cd, zeros,
                  lid_acc, r1_acc, r2_acc,
                  src_v0, src_v1, dst_v0, dst_v1, rows_v0, rows_v1,
                  semi0, semi1, semg0, semg1, sems0, sems1,
                  accl, acc1, acc2):
    c = lax.axis_index("c")
    s = lax.axis_index("s")
    wid = c * 16 + s

    nl16 = _NL // 16
    nr16 = _NRP // 16
    pltpu.sync_copy(zeros, accl.at[pl.ds(s * nl16, nl16)])
    pltpu.sync_copy(zeros.at[pl.ds(0, nr16)], acc1.at[pl.ds(s * nr16, nr16)])
    pltpu.sync_copy(zeros.at[pl.ds(0, nr16)], acc2.at[pl.ds(s * nr16, nr16)])
    plsc.subcore_barrier()

    common = dict(src_v=(src_v0, src_v1), dst_v=(dst_v0, dst_v1),
                  rows_v=(rows_v0, rows_v1), sem_i=(semi0, semi1),
                  sem_g=(semg0, semg1), sem_s=(sems0, sems1))
    _pipelined_pass(_EL // 32 // _CH, wid * (_EL // 32), lid_ei, lid_ei,
                    t_lid, accl, **common)
    for cs_ref, cd_ref, acc in ((r1cs, r1cd, acc1), (r2cs, r2cd, acc2)):
        _pipelined_pass(_ECP // 32 // _CH, wid * (_ECP // 32), cd_ref, cs_ref,
                        t_lid, acc, **common)

    plsc.subcore_barrier()
    pltpu.sync_copy(accl.at[pl.ds(s * nl16, nl16)], lid_acc.at[c, pl.ds(s * nl16, nl16)])
    pltpu.sync_copy(acc1.at[pl.ds(s * nr16, nr16)], r1_acc.at[c, pl.ds(s * nr16, nr16)])
    pltpu.sync_copy(acc2.at[pl.ds(s * nr16, nr16)], r2_acc.at[c, pl.ds(s * nr16, nr16)])


_NEC = 50
_NRE = 20224
_MS = 1264

@functools.partial(
    pl.kernel,
    out_type=jax.ShapeDtypeStruct((2, 16), jnp.float32),
    scratch_types=[
        pltpu.VMEM((_CH,), jnp.int32),
        pltpu.VMEM((_CH,), jnp.float32),
        pltpu.VMEM((_NRE,), jnp.int32),
        pltpu.VMEM((_MS,), jnp.int32),
        pltpu.VMEM((_NR,), jnp.float32),
        pltpu.VMEM((16,), jnp.float32),
        pltpu.VMEM((16, 16), jnp.float32),
        pltpu.VMEM((1, 16), jnp.float32),
        pltpu.VMEM((1, 16), jnp.float32),
        pltpu.VMEM_SHARED((16, _NRE), jnp.int32),
        pltpu.VMEM_SHARED((_NRE,), jnp.int32),
        pltpu.VMEM_SHARED((16, 16), jnp.float32),
    ],
    mesh=_mesh,
    compiler_params=_sc_params_nl,
)
def _sc_phys_pass(tei_b, md_b, den_b, sl, sr1, sr2, out,
                  src_v, md_v, eid_v, tmp_v, den_v, ovec, ph_v, sa_v, sb_v,
                  eid_sp, merged_sp, phys_sp):
    c = lax.axis_index("c")
    s = lax.axis_index("s")
    lanes = lax.iota(jnp.int32, 16)

    zi = jnp.zeros((16,), jnp.int32)
    def init_body(i, _):
        eid_v[pl.ds(i * 16, 16)] = zi
        return 0
    lax.fori_loop(0, _NRE // 16, init_body, 0)

    def p1_chunk(k):
        pltpu.sync_copy(tei_b.at[c, 0, pl.ds(k * _CH, _CH)], src_v)
        def p1_vreg(j, _):
            idx = src_v[pl.ds(j * 16, 16)]
            my = (k * _CH + j * 16 + 1) + lanes
            plsc.store_scatter(eid_v, [idx], my)
            def rnd(r, _):
                g = plsc.load_gather(eid_v, [idx])
                m = my > g
                @pl.when(jnp.any(m))
                def _():
                    plsc.store_scatter(eid_v, [idx], my, mask=m)
                return 0
            lax.fori_loop(0, 3, rnd, 0)
            return 0
        lax.fori_loop(0, _CH // 16, p1_vreg, 0)

    for r in range((_NEC + 15) // 16):
        k = s + 16 * r
        @pl.when(k < _NEC)
        def _(k=k):
            p1_chunk(k)

    pltpu.sync_copy(eid_v, eid_sp.at[s])
    plsc.subcore_barrier()
    pltpu.sync_copy(eid_sp.at[0, pl.ds(s * _MS, _MS)], tmp_v)
    def cp_body(j, _):
        eid_v[pl.ds(j * 16, 16)] = tmp_v[pl.ds(j * 16, 16)]
        return 0
    lax.fori_loop(0, _MS // 16, cp_body, 0)
    for rr in range(1, 16):
        pltpu.sync_copy(eid_sp.at[rr, pl.ds(s * _MS, _MS)], tmp_v)
        def mx_body(j, _):
            eid_v[pl.ds(j * 16, 16)] = jnp.maximum(
                eid_v[pl.ds(j * 16, 16)], tmp_v[pl.ds(j * 16, 16)])
            return 0
        lax.fori_loop(0, _MS // 16, mx_body, 0)
    pltpu.sync_copy(eid_v.at[pl.ds(0, _MS)], merged_sp.at[pl.ds(s * _MS, _MS)])
    plsc.subcore_barrier()
    pltpu.sync_copy(merged_sp, eid_v)
    pltpu.sync_copy(den_b.at[c], den_v)

    ovec[...] = jnp.zeros((16,), jnp.float32)
    def p2_chunk(k):
        pltpu.sync_copy(tei_b.at[c, 0, pl.ds(k * _CH, _CH)], src_v)
        pltpu.sync_copy(md_b.at[c, pl.ds(k * _CH, _CH)], md_v)
        def p2_vreg(j, acc):
            idx = src_v[pl.ds(j * 16, 16)]
            my = (k * _CH + j * 16 + 1) + lanes
            g = plsc.load_gather(eid_v, [idx])
            w = g == my
            dg = plsc.load_gather(den_v, [idx])
            mdv = md_v[pl.ds(j * 16, 16)]
            return acc + jnp.where(w, mdv / dg, 0.0)
        acc = lax.fori_loop(0, _CH // 16, p2_vreg, jnp.zeros((16,), jnp.float32))
        ovec[...] += acc

    for r in range((_NEC + 15) // 16):
        k = s + 16 * r
        @pl.when(k < _NEC)
        def _(k=k):
            p2_chunk(k)

    pltpu.sync_copy(ovec, phys_sp.at[s])
    plsc.subcore_barrier()

    @pl.when(s == 0)
    def _():
        pltpu.sync_copy(phys_sp, ph_v)
        phs = jnp.zeros((16,), jnp.float32)
        for rr in range(16):
            phs = phs + ph_v[rr]
        def bsum(v):
            return jnp.broadcast_to(jnp.sum(v), (16,))
        nl = jnp.full((16,), float(_NL), jnp.float32)
        nr = jnp.full((16,), float(_NR), jnp.float32)

        @pl.when(c == 0)
        def _():
            pltpu.sync_copy(sl, sa_v)
            pltpu.sync_copy(sr1, sb_v)
            tv = bsum(sa_v[0]) / nl + (bsum(sb_v[0]) + bsum(phs)) / nr
            ovec[...] = jnp.where(lanes == 0, tv, 0.0)
            pltpu.sync_copy(ovec, out.at[0])

        @pl.when(c == 1)
        def _():
            pltpu.sync_copy(sr2, sb_v)
            tv = (bsum(sb_v[0]) + bsum(phs)) / nr
            ovec[...] = jnp.where(lanes == 0, tv, 0.0)
            pltpu.sync_copy(ovec, out.at[1])


def _k0_body(lpos_ref, lx_ref, rpos_ref, rx_ref, rb_ref, dt_ref,
             tlid_ref, trad_ref):
    pos = lpos_ref[...]
    x2 = lx_ref[:, 2:3]
    sq = jnp.sum(pos * pos, axis=1, keepdims=True)
    ones = jnp.ones_like(x2)
    z = jnp.zeros_like(pos[:, 0:2])
    tlid_ref[...] = jnp.concatenate([pos, x2, sq, ones, z], axis=1)

    rpos = rpos_ref[...]
    sp = jnp.abs(rx_ref[:, 2:3])
    b = rb_ref[...]
    nd = jnp.zeros_like(sp)
    for bb in range(8):
        nd = jnp.where(b == bb, dt_ref[0:1, bb:bb + 1], nd)
    nd = jnp.maximum(nd, 0.01)
    rz = jnp.zeros_like(rpos)
    trad_ref[...] = jnp.concatenate([rpos, sp, nd, rz], axis=1)


def _b_radar(a0, a1, p, ro, ndt):
    acc = jnp.transpose(a0 + a1)
    s2 = acc[0:3]
    s1 = acc[4:5]
    cnt = acc[5:6]
    rsq = jnp.sum(p * p, axis=0, keepdims=True)
    sum_d = cnt * rsq - 2.0 * jnp.sum(p * s2, axis=0, keepdims=True) + s1
    val = sum_d / jnp.maximum(cnt, 1.0) ** 2
    spat = jnp.where(cnt > 0, val, _GHOST)
    lv = jnp.clip(ro, _R_MIN, _R_MAX)
    den = 2.0 * jnp.exp(lv) * ndt * ndt + 1e-9
    return den, jnp.sum(spat / den + 0.5 * lv)


def _b_mind2(gs, gd, g, b2):
    gst = jnp.transpose(gs)
    gdt = jnp.transpose(gd)
    ps = gst[0:3]
    sp = gst[3:4]
    nd = gst[4:5]
    pd = gdt[0:3]
    mv = pd - ps
    nrm = jnp.sqrt(jnp.sum(mv * mv, axis=0, keepdims=True))
    unit = mv / (nrm + 1e-9)
    pred = ps + sp * unit * nd
    a2 = jnp.sum(pred * pred, axis=0, keepdims=True)
    d2 = jnp.maximum(
        a2 + b2 - 2.0 * jnp.dot(g, pred, preferred_element_type=jnp.float32),
        0.0)
    return jnp.min(d2, axis=0)


def _bmd_body(g1s_ref, g1d_ref, g2s_ref, g2d_ref, gt_ref, md1_ref, md2_ref):
    g = gt_ref[...]
    b2 = jnp.sum(g * g, axis=1, keepdims=True)
    md1_ref[...] = _b_mind2(g1s_ref[...], g1d_ref[...], g, b2)
    md2_ref[...] = _b_mind2(g2s_ref[...], g2d_ref[...], g, b2)


def _bnode_body(la0_ref, la1_ref, lpos_ref, lx_ref, lo_ref,
                r1a0_ref, r1a1_ref, r1pos_ref, r1o_ref, tr1_ref,
                r2a0_ref, r2a1_ref, r2pos_ref, r2o_ref, tr2_ref,
                den1_ref, den2_ref, sl_ref, s1_ref, s2_ref):
    i = pl.program_id(0)

    acc = jnp.transpose(la0_ref[0] + la1_ref[0])
    cnt = acc[5:6]
    dc = jnp.maximum(cnt, 1.0)
    mp = acc[0:3] / dc
    mi = acc[3:4] / dc
    p = jnp.transpose(lpos_ref[...])
    res_pos = jnp.sum((p - mp) ** 2, axis=0, keepdims=True)
    res_int = (jnp.transpose(lx_ref[...])[2:3] - mi) ** 2
    lv = jnp.clip(jnp.transpose(lo_ref[...]), _L_MIN, _L_MAX)
    prec = jnp.exp(-lv)
    tot_l = jnp.sum(0.5 * prec * res_pos + 0.5 * prec * res_int + 0.5 * lv)

    den1, tot_1 = _b_radar(r1a0_ref[0], r1a1_ref[0],
                           jnp.transpose(r1pos_ref[...]),
                           jnp.transpose(r1o_ref[...]),
                           jnp.transpose(tr1_ref[...])[4:5])
    den2, tot_2 = _b_radar(r2a0_ref[0], r2a1_ref[0],
                           jnp.transpose(r2pos_ref[...]),
                           jnp.transpose(r2o_ref[...]),
                           jnp.transpose(tr2_ref[...])[4:5])
    den1_ref[...] = jnp.transpose(den1)
    den2_ref[...] = jnp.transpose(den2)

    iota16 = lax.broadcasted_iota(jnp.int32, (1, 16), 1)
    @pl.when(i == 0)
    def _():
        sl_ref[...] = jnp.zeros((1, 16), jnp.float32)
        s1_ref[...] = jnp.zeros((1, 16), jnp.float32)
        s2_ref[...] = jnp.zeros((1, 16), jnp.float32)
    sl_ref[...] += jnp.where(iota16 == 0, tot_l, 0.0)
    s1_ref[...] += jnp.where(iota16 == 0, tot_1, 0.0)
    s2_ref[...] += jnp.where(iota16 == 0, tot_2, 0.0)


def _blk(shape, imap):
    return pl.BlockSpec(shape, imap)


def kernel(lidar_out, lidar_pos, lidar_x, lidar_spatial_edge_index, radar1_out,
           radar1_pos, radar1_x, radar1_batch, radar1_temporal_edge_index,
           radar1_to_lidar_src, radar1_to_lidar_dst, radar2_out, radar2_pos,
           radar2_x, radar2_batch, radar2_temporal_edge_index,
           radar2_to_lidar_src, radar2_to_lidar_dst, dt_sec, gt_radar_pos):
    f32, i32 = jnp.float32, jnp.int32

    lid_ei = lidar_spatial_edge_index.astype(i32)
    pad_c = _ECP - 200000
    pad_t = _ETP - 100000
    cpad_s = _NR + (jnp.arange(pad_c, dtype=i32) % (_NRP - _NR))
    cpad_d = jnp.arange(pad_c, dtype=i32) % _NL
    r1cs = jnp.concatenate([radar1_to_lidar_src.astype(i32), cpad_s])
    r1cd = jnp.concatenate([radar1_to_lidar_dst.astype(i32), cpad_d])
    r2cs = jnp.concatenate([radar2_to_lidar_src.astype(i32), cpad_s])
    r2cd = jnp.concatenate([radar2_to_lidar_dst.astype(i32), cpad_d])
    tpad = jnp.broadcast_to(jnp.arange(pad_t, dtype=i32) % _NR, (2, pad_t))
    tei1 = jnp.concatenate([radar1_temporal_edge_index.astype(i32), tpad], axis=1)
    tei2 = jnp.concatenate([radar2_temporal_edge_index.astype(i32), tpad], axis=1)
    zeros = jnp.zeros((_NL // 16, _D), f32)
    dt2 = dt_sec.reshape(1, 8).astype(f32)

    rpos = jnp.concatenate([radar1_pos, radar2_pos], axis=0)
    rx = jnp.concatenate([radar1_x, radar2_x], axis=0)
    rb = jnp.concatenate([radar1_batch.astype(i32),
                          radar2_batch.astype(i32)]).reshape(2 * _NR, 1)
    t_lid, t_rad = pl.pallas_call(
        _k0_body,
        grid=(100,),
        in_specs=[_blk((1000, 3), lambda i: (i, 0)),
                  _blk((1000, 3), lambda i: (i, 0)),
                  _blk((400, 3), lambda i: (i, 0)),
                  _blk((400, 3), lambda i: (i, 0)),
                  _blk((400, 1), lambda i: (i, 0)),
                  _blk((1, 8), lambda i: (0, 0))],
        out_specs=[_blk((1000, _D), lambda i: (i, 0)),
                   _blk((400, _D), lambda i: (i, 0))],
        out_shape=[jax.ShapeDtypeStruct((_NL, _D), f32),
                   jax.ShapeDtypeStruct((2 * _NR, _D), f32)],
    )(lidar_pos, lidar_x, rpos, rx, rb, dt2)
    t_r1 = t_rad[:_NR]
    t_r2 = t_rad[_NR:]

    g1s, g1d, g2s, g2d = _sc_temporal_pass(t_r1, t_r2, tei1, tei2)

    lid_acc, r1_acc, r2_acc = _sc_edge_pass(
        t_lid, lid_ei, r1cs, r1cd, r2cs, r2cd, zeros)

    LB, RB, TB = 800, 160, 1024
    md1, md2 = pl.pallas_call(
        _bmd_body,
        grid=(125,),
        in_specs=[_blk((TB, _D), lambda i: (i, 0)),
                  _blk((TB, _D), lambda i: (i, 0)),
                  _blk((TB, _D), lambda i: (i, 0)),
                  _blk((TB, _D), lambda i: (i, 0)),
                  _blk((_NGT, 3), lambda i: (0, 0))],
        out_specs=[_blk((TB,), lambda i: (i,)),
                   _blk((TB,), lambda i: (i,))],
        out_shape=[jax.ShapeDtypeStruct((_ETP,), f32),
                   jax.ShapeDtypeStruct((_ETP,), f32)],
    )(g1s, g1d, g2s, g2d, gt_radar_pos)

    (den1, den2, s_lid, s_r1, s_r2) = pl.pallas_call(
        _bnode_body,
        grid=(125,),
        in_specs=[_blk((1, LB, _D), lambda i: (0, i, 0)),
                  _blk((1, LB, _D), lambda i: (1, i, 0)),
                  _blk((LB, 3), lambda i: (i, 0)),
                  _blk((LB, 3), lambda i: (i, 0)),
                  _blk((LB, 1), lambda i: (i, 0)),
                  _blk((1, RB, _D), lambda i: (0, i, 0)),
                  _blk((1, RB, _D), lambda i: (1, i, 0)),
                  _blk((RB, 3), lambda i: (i, 0)),
                  _blk((RB, 1), lambda i: (i, 0)),
                  _blk((RB, _D), lambda i: (i, 0)),
                  _blk((1, RB, _D), lambda i: (0, i, 0)),
                  _blk((1, RB, _D), lambda i: (1, i, 0)),
                  _blk((RB, 3), lambda i: (i, 0)),
                  _blk((RB, 1), lambda i: (i, 0)),
                  _blk((RB, _D), lambda i: (i, 0))],
        out_specs=[_blk((RB, 1), lambda i: (i, 0)),
                   _blk((RB, 1), lambda i: (i, 0)),
                   _blk((1, 16), lambda i: (0, 0)),
                   _blk((1, 16), lambda i: (0, 0)),
                   _blk((1, 16), lambda i: (0, 0))],
        out_shape=[jax.ShapeDtypeStruct((_NR, 1), f32),
                   jax.ShapeDtypeStruct((_NR, 1), f32),
                   jax.ShapeDtypeStruct((1, 16), f32),
                   jax.ShapeDtypeStruct((1, 16), f32),
                   jax.ShapeDtypeStruct((1, 16), f32)],
    )(lid_acc, lid_acc, lidar_pos, lidar_x, lidar_out,
      r1_acc, r1_acc, radar1_pos, radar1_out, t_r1,
      r2_acc, r2_acc, radar2_pos, radar2_out, t_r2)

    tei_b = jnp.stack([tei1, tei2])
    md_b = jnp.stack([md1, md2])
    den_b = jnp.stack([den1.reshape(_NR), den2.reshape(_NR)])
    tot = _sc_phys_pass(tei_b, md_b, den_b, s_lid, s_r1, s_r2)
    return tot[0, 0] + tot[1, 0]
